# baseline (speedup 1.0000x reference)
import jax
import jax.numpy as jnp
from jax import lax
from jax.experimental import pallas as pl
from jax.experimental.pallas import tpu as pltpu

_N = 4096
_F = 128
_HEADS = 2
_HID = 32
_OUT = _HEADS * _HID

_BA = 512


def _fused_kernel(vert_ref, A_ref, Wt_ref, edge_ref, out_ref,
                  ge0_ref, ge1_ref, edst_ref, et_ref):
    f32 = jnp.float32
    bf16 = jnp.bfloat16
    i = pl.program_id(0)

    @pl.when(i == 0)
    def _project():
        g = lax.dot_general(vert_ref[...], Wt_ref[...],
                            (((1,), (1,)), ((), ())),
                            preferred_element_type=f32)
        scores = jnp.exp(jnp.dot(g, A_ref[...], preferred_element_type=f32))
        edst_ref[...] = scores[:, 0:4].astype(bf16)
        et_ref[...] = scores[:, 4:8].T.astype(bf16)
        iota = lax.broadcasted_iota(jnp.int32, (_N, _HID), 1)
        onescol = (iota == 0).astype(bf16)
        ge0_ref[:, 0:_HID] = g[:, 0:_HID].astype(bf16)
        ge0_ref[:, _HID:2 * _HID] = onescol
        ge1_ref[:, 0:_HID] = g[:, _HID:2 * _HID].astype(bf16)
        ge1_ref[:, _HID:2 * _HID] = onescol

    mask = edge_ref[...] != 0
    tiny = jnp.bfloat16(1e-30)
    row = pl.ds(i * _BA, _BA)
    for h, ge_ref in ((0, ge0_ref), (1, ge1_ref)):
        el1 = edst_ref[row, h:h + 1]
        el2 = edst_ref[row, 2 + h:3 + h]
        p1 = el1 * et_ref[h:h + 1, :]
        p2 = el2 * et_ref[2 + h:3 + h, :]
        p = jnp.where(mask, jnp.maximum(p1, p2), tiny)
        r = jnp.dot(p, ge_ref[...], preferred_element_type=f32)
        rt = r.T
        inv = jnp.float32(1.0) / rt[_HID:_HID + 1, :]
        o = rt[0:_HID, :] * inv
        out_ref[h * _HID:(h + 1) * _HID, :] = jnp.where(
            o > 0, o, jnp.exp(o) - jnp.float32(1.0))


def kernel(vert, edge, W, a_l, a_r):
    f32 = jnp.float32
    fifth = jnp.float32(0.2)
    C = jnp.stack([a_l, fifth * a_l, a_r, fifth * a_r], axis=1).astype(f32)
    A = jnp.einsum('ij,fc->ifcj', jnp.eye(_HEADS, dtype=f32), C).reshape(
        _OUT, 8)
    edge8 = edge.view(jnp.int8)

    out_t = pl.pallas_call(
        _fused_kernel,
        grid=(_N // _BA,),
        in_specs=[
            pl.BlockSpec((_N, _F), lambda i: (0, 0)),
            pl.BlockSpec((_OUT, 8), lambda i: (0, 0)),
            pl.BlockSpec((_OUT, _F), lambda i: (0, 0)),
            pl.BlockSpec((_BA, _N), lambda i: (i, 0)),
        ],
        out_specs=pl.BlockSpec((_OUT, _BA), lambda i: (0, i)),
        out_shape=jax.ShapeDtypeStruct((_OUT, _N), f32),
        scratch_shapes=[
            pltpu.VMEM((_N, 2 * _HID), jnp.bfloat16),
            pltpu.VMEM((_N, 2 * _HID), jnp.bfloat16),
            pltpu.VMEM((_N, 4), jnp.bfloat16),
            pltpu.VMEM((4, _N), jnp.bfloat16),
        ],
    )(vert, A, W.T, edge8)
    return out_t.T


# speedup vs baseline: 1.0162x; 1.0162x over previous
import jax
import jax.numpy as jnp
from jax import lax
from jax.experimental import pallas as pl
from jax.experimental.pallas import tpu as pltpu

_N = 4096
_F = 128
_HEADS = 2
_HID = 32
_OUT = _HEADS * _HID

_BA = 1024


def _fused_kernel(vert_ref, A_ref, Wt_ref, edge_ref, out_ref,
                  ge0_ref, ge1_ref, edst_ref, et_ref):
    f32 = jnp.float32
    bf16 = jnp.bfloat16
    i = pl.program_id(0)

    @pl.when(i == 0)
    def _project():
        g = lax.dot_general(vert_ref[...], Wt_ref[...],
                            (((1,), (1,)), ((), ())),
                            preferred_element_type=f32)
        scores = jnp.exp(jnp.dot(g, A_ref[...], preferred_element_type=f32))
        edst_ref[...] = scores[:, 0:4].astype(bf16)
        et_ref[...] = scores[:, 4:8].T.astype(bf16)
        iota = lax.broadcasted_iota(jnp.int32, (_N, _HID), 1)
        onescol = (iota == 0).astype(bf16)
        ge0_ref[:, 0:_HID] = g[:, 0:_HID].astype(bf16)
        ge0_ref[:, _HID:2 * _HID] = onescol
        ge1_ref[:, 0:_HID] = g[:, _HID:2 * _HID].astype(bf16)
        ge1_ref[:, _HID:2 * _HID] = onescol

    mask = edge_ref[...] != 0
    tiny = jnp.bfloat16(1e-30)
    row = pl.ds(i * _BA, _BA)
    for h, ge_ref in ((0, ge0_ref), (1, ge1_ref)):
        el1 = edst_ref[row, h:h + 1]
        el2 = edst_ref[row, 2 + h:3 + h]
        p1 = el1 * et_ref[h:h + 1, :]
        p2 = el2 * et_ref[2 + h:3 + h, :]
        p = jnp.where(mask, jnp.maximum(p1, p2), tiny)
        r = jnp.dot(p, ge_ref[...], preferred_element_type=f32)
        rt = r.T
        inv = jnp.float32(1.0) / rt[_HID:_HID + 1, :]
        o = rt[0:_HID, :] * inv
        out_ref[h * _HID:(h + 1) * _HID, :] = jnp.where(
            o > 0, o, jnp.exp(o) - jnp.float32(1.0))


def kernel(vert, edge, W, a_l, a_r):
    f32 = jnp.float32
    fifth = jnp.float32(0.2)
    C = jnp.stack([a_l, fifth * a_l, a_r, fifth * a_r], axis=1).astype(f32)
    A = jnp.einsum('ij,fc->ifcj', jnp.eye(_HEADS, dtype=f32), C).reshape(
        _OUT, 8)
    edge8 = edge.view(jnp.int8)

    out_t = pl.pallas_call(
        _fused_kernel,
        grid=(_N // _BA,),
        in_specs=[
            pl.BlockSpec((_N, _F), lambda i: (0, 0)),
            pl.BlockSpec((_OUT, 8), lambda i: (0, 0)),
            pl.BlockSpec((_OUT, _F), lambda i: (0, 0)),
            pl.BlockSpec((_BA, _N), lambda i: (i, 0)),
        ],
        out_specs=pl.BlockSpec((_OUT, _BA), lambda i: (0, i)),
        out_shape=jax.ShapeDtypeStruct((_OUT, _N), f32),
        scratch_shapes=[
            pltpu.VMEM((_N, 2 * _HID), jnp.bfloat16),
            pltpu.VMEM((_N, 2 * _HID), jnp.bfloat16),
            pltpu.VMEM((_N, 4), jnp.bfloat16),
            pltpu.VMEM((4, _N), jnp.bfloat16),
        ],
    )(vert, A, W.T, edge8)
    return out_t.T
